# Initial kernel scaffold; baseline (speedup 1.0000x reference)
import functools

import jax
import jax.numpy as jnp
from jax import lax
from jax.experimental import pallas as pl
from jax.experimental.pallas import tpu as pltpu
from jax.experimental.pallas import tpu_sc as plsc

N = 10000
NP = 10240
E = 320000
F = 128
D = 16
G = 64
NC = 2
NS = 16
NW = NC * NS
CH = 128
EPT = 10240
NCH = EPT // CH
EPAD = NW * EPT
STRIPE = NP // NS


def _mm1_body(x_ref, w_ref, o_ref):
    h = jnp.dot(x_ref[...], w_ref[...], preferred_element_type=jnp.float32)
    o_ref[0:N, :] = h
    o_ref[N:NP, :] = jnp.zeros((NP - N, D), jnp.float32)


_mm1 = pl.pallas_call(
    _mm1_body,
    out_shape=jax.ShapeDtypeStruct((NP, D), jnp.float32),
)


def _mm2_body(p_ref, w_ref, o_ref):
    a = jax.nn.relu(p_ref[0:NP, :] + p_ref[NP:2 * NP, :])
    o_ref[...] = jnp.dot(a, w_ref[...], preferred_element_type=jnp.float32)


_mm2 = pl.pallas_call(
    _mm2_body,
    out_shape=jax.ShapeDtypeStruct((NP, D), jnp.float32),
)


def _final_body(p_ref, batch_ref, wfc_ref, o_ref):
    a = p_ref[0:N, :] + p_ref[NP:NP + N, :]
    b = batch_ref[...]
    onehot = (b == lax.broadcasted_iota(jnp.int32, (1, G), 1)).astype(jnp.float32)
    sums = lax.dot_general(onehot, a, (((0,), (0,)), ((), ())),
                           preferred_element_type=jnp.float32)
    cnts = lax.dot_general(onehot, jnp.ones((N, 1), jnp.float32),
                           (((0,), (0,)), ((), ())),
                           preferred_element_type=jnp.float32)
    pooled = sums / jnp.maximum(cnts, 1.0)
    o_ref[...] = jax.nn.sigmoid(
        jnp.dot(pooled, wfc_ref[...], preferred_element_type=jnp.float32))


_final = pl.pallas_call(
    _final_body,
    out_shape=jax.ShapeDtypeStruct((G, 1), jnp.float32),
)


def _scatter_body(h_hbm, src_hbm, dst_hbm, out_hbm,
                  src_v, dst_v, rows_a, rows_b, zero_v, acc, sem_a, sem_b):
    c = lax.axis_index("c")
    s = lax.axis_index("s")
    wid = s * NC + c

    for i in range(CH):
        zero_v[i, :] = jnp.zeros((D,), jnp.float32)
    for k in range(STRIPE // CH):
        pltpu.sync_copy(zero_v, acc.at[pl.ds(s * STRIPE + k * CH, CH)])

    pltpu.sync_copy(src_hbm.at[wid], src_v)
    pltpu.sync_copy(dst_hbm.at[wid], dst_v)
    plsc.subcore_barrier()

    bufs = (rows_a, rows_b)
    sems = (sem_a, sem_b)
    desc = pltpu.async_copy(h_hbm.at[src_v.at[0]], rows_a, sem_a)
    for j in range(NCH):
        cur = bufs[j % 2]
        desc.wait()
        if j + 1 < NCH:
            desc = pltpu.async_copy(h_hbm.at[src_v.at[j + 1]],
                                    bufs[(j + 1) % 2], sems[(j + 1) % 2])
        pltpu.sync_copy(cur, acc.at[dst_v.at[j]], add=True)

    plsc.subcore_barrier()
    pltpu.sync_copy(acc.at[pl.ds(s * STRIPE, STRIPE)],
                    out_hbm.at[pl.ds(c * NP + s * STRIPE, STRIPE)])


_scatter = functools.partial(
    pl.kernel,
    out_type=jax.ShapeDtypeStruct((NC * NP, D), jnp.float32),
    mesh=plsc.VectorSubcoreMesh(core_axis_name="c", subcore_axis_name="s"),
    scratch_types=[
        pltpu.VMEM((NCH, CH), jnp.int32),
        pltpu.VMEM((NCH, CH), jnp.int32),
        pltpu.VMEM((CH, D), jnp.float32),
        pltpu.VMEM((CH, D), jnp.float32),
        pltpu.VMEM((CH, D), jnp.float32),
        pltpu.VMEM_SHARED((NP, D), jnp.float32),
        pltpu.SemaphoreType.DMA,
        pltpu.SemaphoreType.DMA,
    ],
)(_scatter_body)


def kernel(x, edge_index, batch, W1, W2, Wfc):
    src = edge_index[0]
    dst = edge_index[1]
    pad = EPAD - E
    padidx = N + (jnp.arange(pad, dtype=jnp.int32) % (NP - N))
    src_p = jnp.concatenate([src, padidx]).reshape(NW, NCH, CH)
    dst_p = jnp.concatenate([dst, padidx]).reshape(NW, NCH, CH)

    h1 = _mm1(x, W1)
    p1 = _scatter(h1, src_p, dst_p)
    h2 = _mm2(p1, W2)
    p2 = _scatter(h2, src_p, dst_p)
    return _final(p2, batch.reshape(N, 1), Wfc)


# speedup vs baseline: 15.8431x; 15.8431x over previous
import functools

import jax
import jax.numpy as jnp
from jax import lax
from jax.experimental import pallas as pl
from jax.experimental.pallas import tpu as pltpu
from jax.experimental.pallas import tpu_sc as plsc

N = 10000
NP = 10240
E = 320000
F = 128
D = 16
G = 64
NC = 2
NS = 16
NW = NC * NS
CH = 128
EPT = 10240
NCH = EPT // CH
EPAD = NW * EPT
STRIPE = NP // NS


def _mm1_body(x_ref, w_ref, o_ref):
    h = jnp.dot(x_ref[...], w_ref[...], preferred_element_type=jnp.float32)
    o_ref[0:N, :] = h
    o_ref[N:NP, :] = jnp.zeros((NP - N, D), jnp.float32)


_mm1 = pl.pallas_call(
    _mm1_body,
    out_shape=jax.ShapeDtypeStruct((NP, D), jnp.float32),
)


def _mm2_body(p_ref, w_ref, o_ref):
    a = jax.nn.relu(p_ref[0:NP, :] + p_ref[NP:2 * NP, :])
    o_ref[...] = jnp.dot(a, w_ref[...], preferred_element_type=jnp.float32)


_mm2 = pl.pallas_call(
    _mm2_body,
    out_shape=jax.ShapeDtypeStruct((NP, D), jnp.float32),
)


def _final_body(p_ref, batch_ref, wfc_ref, o_ref):
    a = p_ref[0:N, :] + p_ref[NP:NP + N, :]
    b = batch_ref[...]
    onehot = (b == lax.broadcasted_iota(jnp.int32, (1, G), 1)).astype(jnp.float32)
    sums = lax.dot_general(onehot, a, (((0,), (0,)), ((), ())),
                           preferred_element_type=jnp.float32)
    cnts = lax.dot_general(onehot, jnp.ones((N, 1), jnp.float32),
                           (((0,), (0,)), ((), ())),
                           preferred_element_type=jnp.float32)
    pooled = sums / jnp.maximum(cnts, 1.0)
    o_ref[...] = jax.nn.sigmoid(
        jnp.dot(pooled, wfc_ref[...], preferred_element_type=jnp.float32))


_final = pl.pallas_call(
    _final_body,
    out_shape=jax.ShapeDtypeStruct((G, 1), jnp.float32),
)


def _scatter_body(h_hbm, src_hbm, dst_hbm, out_hbm,
                  src_v, dst_v, rows_a, rows_b, zero_v, acc, sem_a, sem_b):
    c = lax.axis_index("c")
    s = lax.axis_index("s")
    wid = s * NC + c

    for i in range(CH):
        zero_v[i, :] = jnp.zeros((D,), jnp.float32)
    for k in range(STRIPE // CH):
        pltpu.sync_copy(zero_v, acc.at[pl.ds(s * STRIPE + k * CH, CH)])

    pltpu.sync_copy(src_hbm.at[wid], src_v)
    pltpu.sync_copy(dst_hbm.at[wid], dst_v)
    plsc.subcore_barrier()

    bufs = (rows_a, rows_b)
    sems = (sem_a, sem_b)
    desc = pltpu.async_copy(h_hbm.at[src_v.at[0]], rows_a, sem_a)
    for j in range(NCH):
        cur = bufs[j % 2]
        desc.wait()
        if j + 1 < NCH:
            desc = pltpu.async_copy(h_hbm.at[src_v.at[j + 1]],
                                    bufs[(j + 1) % 2], sems[(j + 1) % 2])
        pltpu.sync_copy(cur, acc.at[dst_v.at[j]], add=True)

    plsc.subcore_barrier()
    pltpu.sync_copy(acc.at[pl.ds(s * STRIPE, STRIPE)],
                    out_hbm.at[pl.ds(c * NP + s * STRIPE, STRIPE)])


@functools.cache
def _scatter():
    return functools.partial(
        pl.kernel,
        out_type=jax.ShapeDtypeStruct((NC * NP, D), jnp.float32),
        mesh=plsc.VectorSubcoreMesh(core_axis_name="c", subcore_axis_name="s",
                                    num_cores=NC, num_subcores=NS),
        scratch_types=[
            pltpu.VMEM((NCH, CH), jnp.int32),
            pltpu.VMEM((NCH, CH), jnp.int32),
            pltpu.VMEM((CH, D), jnp.float32),
            pltpu.VMEM((CH, D), jnp.float32),
            pltpu.VMEM((CH, D), jnp.float32),
            pltpu.VMEM_SHARED((NP, D), jnp.float32),
            pltpu.SemaphoreType.DMA,
            pltpu.SemaphoreType.DMA,
        ],
        compiler_params=pltpu.CompilerParams(use_tc_tiling_on_sc=False),
    )(_scatter_body)


def kernel(x, edge_index, batch, W1, W2, Wfc):
    src = edge_index[0]
    dst = edge_index[1]
    pad = EPAD - E
    padidx = N + (jnp.arange(pad, dtype=jnp.int32) % (NP - N))
    src_p = jnp.concatenate([src, padidx]).reshape(NW, NCH, CH)
    dst_p = jnp.concatenate([dst, padidx]).reshape(NW, NCH, CH)

    scatter = _scatter()
    h1 = _mm1(x, W1)
    p1 = scatter(h1, src_p, dst_p)
    h2 = _mm2(p1, W2)
    p2 = scatter(h2, src_p, dst_p)
    return _final(p2, batch.reshape(N, 1), Wfc)


# speedup vs baseline: 27.1083x; 1.7111x over previous
import functools

import jax
import jax.numpy as jnp
from jax import lax
from jax.experimental import pallas as pl
from jax.experimental.pallas import tpu as pltpu
from jax.experimental.pallas import tpu_sc as plsc

N = 10000
NP = 10240
E = 320000
F = 128
D = 16
G = 64
NC = 2
NS = 16
NW = NC * NS
CH = 128
EPT = 10240
NCH = EPT // CH
EPAD = NW * EPT
STRIPE = NP // NS


def _mm1_body(x_ref, w_ref, o_ref):
    h = jnp.dot(x_ref[...], w_ref[...], preferred_element_type=jnp.float32)
    o_ref[0:N, :] = h
    o_ref[N:NP, :] = jnp.zeros((NP - N, D), jnp.float32)


_mm1 = pl.pallas_call(
    _mm1_body,
    out_shape=jax.ShapeDtypeStruct((NP, D), jnp.float32),
)


def _mm2_body(p_ref, w_ref, o_ref):
    a = jax.nn.relu(p_ref[0:NP, :] + p_ref[NP:2 * NP, :])
    o_ref[...] = jnp.dot(a, w_ref[...], preferred_element_type=jnp.float32)


_mm2 = pl.pallas_call(
    _mm2_body,
    out_shape=jax.ShapeDtypeStruct((NP, D), jnp.float32),
)


def _final_body(p_ref, batch_ref, wfc_ref, o_ref):
    a = p_ref[0:N, :] + p_ref[NP:NP + N, :]
    b = batch_ref[...]
    onehot = (b == lax.broadcasted_iota(jnp.int32, (1, G), 1)).astype(jnp.float32)
    sums = lax.dot_general(onehot, a, (((0,), (0,)), ((), ())),
                           preferred_element_type=jnp.float32)
    cnts = lax.dot_general(onehot, jnp.ones((N, 1), jnp.float32),
                           (((0,), (0,)), ((), ())),
                           preferred_element_type=jnp.float32)
    pooled = sums / jnp.maximum(cnts, 1.0)
    o_ref[...] = jax.nn.sigmoid(
        jnp.dot(pooled, wfc_ref[...], preferred_element_type=jnp.float32))


_final = pl.pallas_call(
    _final_body,
    out_shape=jax.ShapeDtypeStruct((G, 1), jnp.float32),
)


NBUF = 8
LAG = 4


def _scatter_body(h_hbm, src_hbm, dst_hbm, out_hbm,
                  src_v, dst_v, bufs, zero_v, acc, gsems, ssems):
    c = lax.axis_index("c")
    s = lax.axis_index("s")
    wid = s * NC + c

    for i in range(CH):
        zero_v[i, :] = jnp.zeros((D,), jnp.float32)
    for k in range(STRIPE // CH):
        pltpu.sync_copy(zero_v, acc.at[pl.ds(s * STRIPE + k * CH, CH)])

    pltpu.sync_copy(src_hbm.at[wid], src_v)
    pltpu.sync_copy(dst_hbm.at[wid], dst_v)
    plsc.subcore_barrier()

    gd = [None] * NBUF
    sd = [None] * NBUF
    for t in range(NCH + LAG):
        if t < NCH:
            b = t % NBUF
            if t >= NBUF:
                sd[b].wait()
            gd[b] = pltpu.async_copy(h_hbm.at[src_v.at[t]], bufs[b],
                                     gsems.at[b])
        u = t - LAG
        if u >= 0:
            bu = u % NBUF
            gd[bu].wait()
            sd[bu] = pltpu.async_copy(bufs[bu], acc.at[dst_v.at[u]],
                                      ssems.at[bu], add=True)
    for b in range(NBUF):
        sd[b].wait()

    plsc.subcore_barrier()
    pltpu.sync_copy(acc.at[pl.ds(s * STRIPE, STRIPE)],
                    out_hbm.at[pl.ds(c * NP + s * STRIPE, STRIPE)])


@functools.cache
def _scatter():
    return functools.partial(
        pl.kernel,
        out_type=jax.ShapeDtypeStruct((NC * NP, D), jnp.float32),
        mesh=plsc.VectorSubcoreMesh(core_axis_name="c", subcore_axis_name="s",
                                    num_cores=NC, num_subcores=NS),
        scratch_types=[
            pltpu.VMEM((NCH, CH), jnp.int32),
            pltpu.VMEM((NCH, CH), jnp.int32),
            [pltpu.VMEM((CH, D), jnp.float32) for _ in range(NBUF)],
            pltpu.VMEM((CH, D), jnp.float32),
            pltpu.VMEM_SHARED((NP, D), jnp.float32),
            pltpu.SemaphoreType.DMA((NBUF,)),
            pltpu.SemaphoreType.DMA((NBUF,)),
        ],
        compiler_params=pltpu.CompilerParams(use_tc_tiling_on_sc=False),
    )(_scatter_body)


def kernel(x, edge_index, batch, W1, W2, Wfc):
    src = edge_index[0]
    dst = edge_index[1]
    pad = EPAD - E
    padidx = N + (jnp.arange(pad, dtype=jnp.int32) % (NP - N))
    src_p = jnp.concatenate([src, padidx]).reshape(NW, NCH, CH)
    dst_p = jnp.concatenate([dst, padidx]).reshape(NW, NCH, CH)

    scatter = _scatter()
    h1 = _mm1(x, W1)
    p1 = scatter(h1, src_p, dst_p)
    h2 = _mm2(p1, W2)
    p2 = scatter(h2, src_p, dst_p)
    return _final(p2, batch.reshape(N, 1), Wfc)
